# baseline (speedup 1.0000x reference)
import functools

import jax
import jax.numpy as jnp
from jax import lax
from jax.experimental import pallas as pl
from jax.experimental.pallas import tpu as pltpu
from jax.experimental.pallas import tpu_sc as plsc

_B = 16
_C = 64
_HW = 64 * 64
_K = 1024
_N = _B * _HW


def _vq_indices(z_hbm, emb_ref, idx_hbm, loss_ref,
                zbuf, ibuf, in_sem, outi_sem):
    emb = emb_ref[...]
    h = 0.5 * jnp.sum(emb * emb, axis=0)

    def in_copy(i):
        return pltpu.make_async_copy(
            z_hbm.at[i], zbuf.at[i % 2], in_sem.at[i % 2])

    def outi_copy(i):
        return pltpu.make_async_copy(
            ibuf.at[i % 2], idx_hbm.at[i], outi_sem.at[i % 2])

    in_copy(0).start()
    loss_acc = jnp.zeros((_HW,), jnp.float32)
    for i in range(_B):
        if i + 1 < _B:
            in_copy(i + 1).start()
        in_copy(i).wait()
        zb = zbuf[i % 2]
        scores = jax.lax.dot_general(
            emb, zb, (((0,), (0,)), ((), ())),
            preferred_element_type=jnp.float32)
        score = scores - h[:, None]
        idx = jnp.argmax(score, axis=0)
        m = jnp.max(score, axis=0)
        if i >= 2:
            outi_copy(i - 2).wait()
        ibuf[i % 2, 0] = idx
        outi_copy(i).start()
        z_sq = jnp.sum(zb * zb, axis=0)
        loss_acc = loss_acc + (z_sq - 2.0 * m)
    outi_copy(_B - 2).wait()
    outi_copy(_B - 1).wait()
    loss_ref[0] = loss_acc


_INFO = plsc.get_sparse_core_info()
_NW = _INFO.num_cores * _INFO.num_subcores
_PER_W = _N // _NW
_CHUNK = 256
_NCH = _PER_W // _CHUNK


def _sc_gather(table_hbm, idx_hbm, out_hbm, idx_v, rows_v, sem):
    wid = lax.axis_index("s") * _INFO.num_cores + lax.axis_index("c")
    base = wid * _PER_W
    for j in range(_NCH):
        off = base + j * _CHUNK
        pltpu.sync_copy(idx_hbm.at[pl.ds(off, _CHUNK)], idx_v)
        pltpu.async_copy(table_hbm.at[idx_v], rows_v, sem).wait()
        pltpu.sync_copy(rows_v, out_hbm.at[pl.ds(off, _CHUNK)])


@jax.jit
def kernel(z, embedding):
    commitment_cost = 0.25
    z3 = z.reshape(_B, _C, _HW)
    idx, loss_parts = pl.pallas_call(
        _vq_indices,
        in_specs=[
            pl.BlockSpec(memory_space=pl.ANY),
            pl.BlockSpec(memory_space=pltpu.VMEM),
        ],
        out_specs=[
            pl.BlockSpec(memory_space=pl.ANY),
            pl.BlockSpec(memory_space=pltpu.VMEM),
        ],
        out_shape=[
            jax.ShapeDtypeStruct((_B, 1, _HW), jnp.int32),
            jax.ShapeDtypeStruct((1, _HW), jnp.float32),
        ],
        scratch_shapes=[
            pltpu.VMEM((2, _C, _HW), jnp.float32),
            pltpu.VMEM((2, 1, _HW), jnp.int32),
            pltpu.SemaphoreType.DMA((2,)),
            pltpu.SemaphoreType.DMA((2,)),
        ],
    )(z3, embedding)

    mesh = plsc.VectorSubcoreMesh(core_axis_name="c", subcore_axis_name="s")
    gather = functools.partial(
        pl.kernel, mesh=mesh,
        compiler_params=pltpu.CompilerParams(use_tc_tiling_on_sc=False),
        out_type=jax.ShapeDtypeStruct((_N, _C), jnp.float32),
        scratch_types=[
            pltpu.VMEM((_CHUNK,), jnp.int32),
            pltpu.VMEM((_CHUNK, _C), jnp.float32),
            pltpu.SemaphoreType.DMA,
        ],
    )(_sc_gather)
    rows = gather(embedding.T, idx.reshape(_N))

    quantized_out = jnp.transpose(
        rows.reshape(_B, _HW, _C), (0, 2, 1)).reshape(z.shape)
    encoding_indices = idx.reshape(_B, 64, 64)
    loss = (1.0 + commitment_cost) * jnp.sum(loss_parts) / z.size
    return (quantized_out, loss, encoding_indices)


# speedup vs baseline: 1.8197x; 1.8197x over previous
import functools

import jax
import jax.numpy as jnp
from jax.experimental import pallas as pl
from jax.experimental.pallas import tpu as pltpu

_B = 16
_C = 64
_HW = 64 * 64
_K = 1024


def _vq_pipeline(z_hbm, emb_ref, quant_hbm, idx_hbm, loss_ref,
                 zbuf, qbuf, ibuf, in_sem, outq_sem, outi_sem):
    emb = emb_ref[...]
    h = 0.5 * jnp.sum(emb * emb, axis=0)

    def in_copy(i):
        return pltpu.make_async_copy(
            z_hbm.at[i], zbuf.at[i % 2], in_sem.at[i % 2])

    def outq_copy(i):
        return pltpu.make_async_copy(
            qbuf.at[i % 2], quant_hbm.at[i], outq_sem.at[i % 2])

    def outi_copy(i):
        return pltpu.make_async_copy(
            ibuf.at[i % 2], idx_hbm.at[i], outi_sem.at[i % 2])

    in_copy(0).start()
    loss_acc = jnp.zeros((_HW,), jnp.float32)
    for i in range(_B):
        if i + 1 < _B:
            in_copy(i + 1).start()
        in_copy(i).wait()
        zb = zbuf[i % 2]
        scores = jax.lax.dot_general(
            emb, zb, (((0,), (0,)), ((), ())),
            preferred_element_type=jnp.float32)
        score = scores - h[:, None]
        idx = jnp.argmax(score, axis=0)
        onehot = (jax.lax.broadcasted_iota(jnp.int32, (_K, _HW), 0)
                  == idx[None, :]).astype(jnp.float32)
        quant = jax.lax.dot_general(
            emb, onehot, (((1,), (0,)), ((), ())),
            preferred_element_type=jnp.float32)
        if i >= 2:
            outq_copy(i - 2).wait()
            outi_copy(i - 2).wait()
        qbuf[i % 2] = quant
        ibuf[i % 2, 0] = idx
        outq_copy(i).start()
        outi_copy(i).start()
        diff = quant - zb
        loss_acc = loss_acc + jnp.sum(diff * diff, axis=0)
    outq_copy(_B - 2).wait()
    outi_copy(_B - 2).wait()
    outq_copy(_B - 1).wait()
    outi_copy(_B - 1).wait()
    loss_ref[0] = loss_acc


@jax.jit
def kernel(z, embedding):
    commitment_cost = 0.25
    z3 = z.reshape(_B, _C, _HW)
    quant, idx, loss_parts = pl.pallas_call(
        _vq_pipeline,
        in_specs=[
            pl.BlockSpec(memory_space=pl.ANY),
            pl.BlockSpec(memory_space=pltpu.VMEM),
        ],
        out_specs=[
            pl.BlockSpec(memory_space=pl.ANY),
            pl.BlockSpec(memory_space=pl.ANY),
            pl.BlockSpec(memory_space=pltpu.VMEM),
        ],
        out_shape=[
            jax.ShapeDtypeStruct((_B, _C, _HW), jnp.float32),
            jax.ShapeDtypeStruct((_B, 1, _HW), jnp.int32),
            jax.ShapeDtypeStruct((1, _HW), jnp.float32),
        ],
        scratch_shapes=[
            pltpu.VMEM((2, _C, _HW), jnp.float32),
            pltpu.VMEM((2, _C, _HW), jnp.float32),
            pltpu.VMEM((2, 1, _HW), jnp.int32),
            pltpu.SemaphoreType.DMA((2,)),
            pltpu.SemaphoreType.DMA((2,)),
            pltpu.SemaphoreType.DMA((2,)),
        ],
    )(z3, embedding)
    quantized_out = quant.reshape(z.shape)
    encoding_indices = idx.reshape(_B, 64, 64)
    loss = (1.0 + commitment_cost) * jnp.sum(loss_parts) / z.size
    return (quantized_out, loss, encoding_indices)
